# Initial kernel scaffold; baseline (speedup 1.0000x reference)
import jax
import jax.numpy as jnp
from jax.experimental import pallas as pl


def kernel(x, edge_index, batch, W1, b1, W2, b2):
    raise NotImplementedError("write your pallas kernel here")


# speedup vs baseline: 7.7506x; 7.7506x over previous
import functools

import jax
import jax.numpy as jnp
from jax import lax
from jax.experimental import pallas as pl
from jax.experimental.pallas import tpu as pltpu
from jax.experimental.pallas import tpu_sc as plsc

_N = 10000
_E = 320000
_C = 128
_H = 128
_L = 64
_G = 512

_NC = 2
_NS = 16
_CW = 128
_NW = 157
_EP = _NS * _NW * _CW
_NPAD = 10240
_HR = _NPAD // _NC
_ACC1 = _HR + 8
_PR = _NPAD // 2
_HP = _PR // _NC
_ACC2 = _HP + 8
_GACC = _G // _NC + 8
_BW = _NPAD // (_NS * _CW)

_BR = 1280
_BP = 640


def _mesh():
    return plsc.VectorSubcoreMesh(core_axis_name="c", subcore_axis_name="s")


@functools.partial(
    pl.kernel,
    out_type=[jax.ShapeDtypeStruct((_NPAD,), jnp.float32),
              jax.ShapeDtypeStruct((_G,), jnp.float32)],
    mesh=_mesh(),
    scratch_types=[
        pltpu.VMEM((_NW, _CW), jnp.int32),
        pltpu.VMEM((_NW, _CW), jnp.int32),
        pltpu.VMEM((_BW, _CW), jnp.int32),
        pltpu.VMEM((_BW, _CW), jnp.int32),
        pltpu.VMEM((_CW,), jnp.float32),
        pltpu.VMEM_SHARED((_ACC1,), jnp.float32),
        pltpu.VMEM_SHARED((_GACC,), jnp.float32),
    ],
)
def _deg_sc(dst_e, bat_e, deg_out, cnt_out,
            idx_d, idx_l, idx_b, idx_bl, buf, acc, acc_g):
    c = lax.axis_index("c")
    s = lax.axis_index("s")
    pltpu.sync_copy(dst_e.at[s], idx_d)
    pltpu.sync_copy(bat_e.at[s], idx_b)
    rbase = c * _HR
    gbase = c * (_G // _NC)

    def loc(j, carry):
        for g in range(_CW // 16):
            v = idx_d[j, pl.ds(g * 16, 16)] - rbase
            ok = (v >= 0) & (v < _HR)
            idx_l[j, pl.ds(g * 16, 16)] = jnp.where(ok, v, _HR)
        return carry

    lax.fori_loop(0, _NW, loc, 0)

    def locb(j, carry):
        for g in range(_CW // 16):
            v = idx_b[j, pl.ds(g * 16, 16)] - gbase
            ok = (v >= 0) & (v < _G // _NC)
            idx_bl[j, pl.ds(g * 16, 16)] = jnp.where(ok, v, _G // _NC)
        return carry

    lax.fori_loop(0, _BW, locb, 0)

    for g in range(_CW // 16):
        buf[pl.ds(g * 16, 16)] = jnp.zeros((16,), jnp.float32)
    for k in range(3):
        lc = s + 16 * k

        @pl.when(lc < _HR // _CW)
        def _():
            pltpu.sync_copy(buf, acc.at[pl.ds(lc * _CW, _CW)])

    @pl.when(s == 0)
    def _():
        pltpu.sync_copy(buf.at[pl.ds(0, 8)], acc.at[pl.ds(_HR, 8)])
        pltpu.sync_copy(buf, acc_g.at[pl.ds(0, 128)])
        pltpu.sync_copy(buf, acc_g.at[pl.ds(128, 128)])
        pltpu.sync_copy(buf.at[pl.ds(0, 8)], acc_g.at[pl.ds(_G // _NC, 8)])

    for g in range(_CW // 16):
        buf[pl.ds(g * 16, 16)] = jnp.ones((16,), jnp.float32)
    plsc.subcore_barrier()

    def chunk(j, carry):
        pltpu.sync_copy(buf, acc.at[idx_l.at[j]], add=True)
        return carry

    lax.fori_loop(0, _NW, chunk, 0)

    def chunkb(j, carry):
        pltpu.sync_copy(buf, acc_g.at[idx_bl.at[j]], add=True)
        return carry

    lax.fori_loop(0, _BW, chunkb, 0)
    plsc.subcore_barrier()
    for k in range(3):
        lc = s + 16 * k

        @pl.when(lc < _HR // _CW)
        def _():
            pltpu.sync_copy(acc.at[pl.ds(lc * _CW, _CW)], buf)
            pltpu.sync_copy(buf, deg_out.at[pl.ds(c * _HR + lc * _CW, _CW)])

    @pl.when(s < 2)
    def _():
        pltpu.sync_copy(acc_g.at[pl.ds(s * _CW, _CW)], buf)
        pltpu.sync_copy(buf, cnt_out.at[pl.ds(gbase + s * _CW, _CW)])


@functools.partial(
    pl.kernel,
    out_type=jax.ShapeDtypeStruct((_NC, _ACC2, _C), jnp.float32),
    mesh=_mesh(),
    scratch_types=[
        pltpu.VMEM((_NW, _CW), jnp.int32),
        pltpu.VMEM((_NW, _CW), jnp.int32),
        pltpu.VMEM((_NW, _CW), jnp.int32),
        pltpu.VMEM((_NW, _CW), jnp.int32),
        pltpu.VMEM((_CW, _C), jnp.float32),
        pltpu.VMEM_SHARED((_ACC2, _C), jnp.float32),
        pltpu.SemaphoreType.DMA,
    ],
)
def _agg_sc(table2, src_e, dst_e, out, idx_s, idx_d, idx_g, idx_l, rows,
            acc, sem):
    c = lax.axis_index("c")
    s = lax.axis_index("s")
    pltpu.sync_copy(src_e.at[s], idx_s)
    pltpu.sync_copy(dst_e.at[s], idx_d)
    pbase = c * _HP

    def loc(j, carry):
        for g in range(_CW // 16):
            d16 = idx_d[j, pl.ds(g * 16, 16)]
            s16 = idx_s[j, pl.ds(g * 16, 16)]
            hi = d16 >= _PR
            pr = jnp.where(hi, d16 - _PR, d16) - pbase
            ok = (pr >= 0) & (pr < _HP)
            idx_l[j, pl.ds(g * 16, 16)] = jnp.where(ok, pr, _HP)
            idx_g[j, pl.ds(g * 16, 16)] = jnp.where(hi, s16 + _NPAD, s16)
        return carry

    lax.fori_loop(0, _NW, loc, 0)

    def zrow(i, carry):
        for g in range(_C // 16):
            rows[i, pl.ds(g * 16, 16)] = jnp.zeros((16,), jnp.float32)
        return carry

    lax.fori_loop(0, _CW, zrow, 0)
    base = s * (_HP // _NS)
    for off, nrow in ((0, 128), (128, 32)):
        pltpu.sync_copy(rows if nrow == _CW else rows.at[pl.ds(0, nrow)],
                        acc.at[pl.ds(base + off, nrow)])

    @pl.when(s == 0)
    def _():
        pltpu.sync_copy(rows.at[pl.ds(0, 8)], acc.at[pl.ds(_HP, 8)])

    plsc.subcore_barrier()

    def chunk(j, carry):
        pltpu.async_copy(table2.at[idx_g.at[j]], rows, sem).wait()
        pltpu.sync_copy(rows, acc.at[idx_l.at[j]], add=True)
        return carry

    lax.fori_loop(0, _NW, chunk, 0)
    plsc.subcore_barrier()
    for off, nrow in ((0, 128), (128, 32)):
        sl = pl.ds(base + off, nrow)
        stg = rows if nrow == _CW else rows.at[pl.ds(0, nrow)]
        pltpu.sync_copy(acc.at[sl], stg)
        pltpu.sync_copy(stg, out.at[c, sl])


def _half_map(i):
    return (i // 4, i % 4, 0)


def _dinv_of(deg):
    return lax.rsqrt(jnp.maximum(deg + 1.0, 1.0))


def _sel_const(i):
    r = lax.broadcasted_iota(jnp.int32, (_C, _L), 0)
    cc = lax.broadcasted_iota(jnp.int32, (_C, _L), 1)
    p_lo = (r == cc).astype(jnp.float32)
    p_hi = (r == cc + _L).astype(jnp.float32)
    return jnp.where(i < 4, p_lo, p_hi)


def _stage1_body(x_ref, wa_ref, wb_ref, deg_ref, oa_ref, ob_ref):
    dinv = _dinv_of(deg_ref[...])
    dot = functools.partial(jnp.dot, preferred_element_type=jnp.float32,
                            precision=lax.Precision.HIGHEST)
    had = dot(x_ref[...], wa_ref[...]) * dinv
    hbd = dot(x_ref[...], wb_ref[...]) * dinv
    z64 = jnp.zeros((_BR, _C - _L), jnp.float32)
    oa_ref[0] = jnp.concatenate([had, z64], axis=1)
    oa_ref[1] = jnp.concatenate([z64, had], axis=1)
    ob_ref[0] = jnp.concatenate([hbd, z64], axis=1)
    ob_ref[1] = jnp.concatenate([z64, hbd], axis=1)


def _stage1(xp, w1a, w1b, deg2):
    out_sd = jax.ShapeDtypeStruct((2, _NPAD, _C), jnp.float32)
    return pl.pallas_call(
        _stage1_body,
        grid=(_NPAD // _BR,),
        in_specs=[
            pl.BlockSpec((_BR, _C), lambda i: (i, 0)),
            pl.BlockSpec((_C, _L), lambda i: (0, 0)),
            pl.BlockSpec((_C, _L), lambda i: (0, 0)),
            pl.BlockSpec((_BR, 1), lambda i: (i, 0)),
        ],
        out_specs=[pl.BlockSpec((2, _BR, _C), lambda i: (0, i, 0)),
                   pl.BlockSpec((2, _BR, _C), lambda i: (0, i, 0))],
        out_shape=[out_sd, out_sd],
    )(xp, w1a, w1b, deg2)


def _stage2_body(a1a_ref, a1b_ref, ta_ref, tb_ref, deg_ref, b1_ref, w2_ref,
                 o_ref):
    i = pl.program_id(0)
    dot = functools.partial(jnp.dot, preferred_element_type=jnp.float32,
                            precision=lax.Precision.HIGHEST)
    sel = _sel_const(i)
    acc128 = jnp.concatenate(
        [dot(a1a_ref[0], sel), dot(a1b_ref[0], sel)], axis=1)
    t1blk = ta_ref[0] + tb_ref[0]
    dinv = _dinv_of(deg_ref[...])
    pre = dinv * (acc128 + t1blk) + b1_ref[...]
    h = jnp.maximum(pre, 0.0)
    h2d = dot(h, w2_ref[...]) * dinv
    z64 = jnp.zeros((_BR, _C - _L), jnp.float32)
    o_ref[0] = jnp.concatenate([h2d, z64], axis=1)
    o_ref[1] = jnp.concatenate([z64, h2d], axis=1)


def _packed_map(i):
    return ((i % 4) // 2, (i % 4) % 2, 0)


def _stage2(acc1a, acc1b, taba, tabb, deg2, b1, w2):
    return pl.pallas_call(
        _stage2_body,
        grid=(_NPAD // _BR,),
        in_specs=[
            pl.BlockSpec((1, _BR, _C), _packed_map),
            pl.BlockSpec((1, _BR, _C), _packed_map),
            pl.BlockSpec((1, _BR, _C), lambda i: (0, i, 0)),
            pl.BlockSpec((1, _BR, _C), lambda i: (1, i, 0)),
            pl.BlockSpec((_BR, 1), lambda i: (i, 0)),
            pl.BlockSpec((1, _H), lambda i: (0, 0)),
            pl.BlockSpec((_H, _L), lambda i: (0, 0)),
        ],
        out_specs=pl.BlockSpec((2, _BR, _C), lambda i: (0, i, 0)),
        out_shape=jax.ShapeDtypeStruct((2, _NPAD, _C), jnp.float32),
    )(acc1a, acc1b, taba, tabb, deg2, b1, w2)


def _pool_body(a_ref, tlo_ref, thi_ref, dlo_ref, dhi_ref, b2_ref,
               blo_ref, bhi_ref, cnt_ref, o_ref):
    i = pl.program_id(0)
    lane = lax.broadcasted_iota(jnp.int32, (_BP, _C), 1)
    dmix = jnp.where(lane < _L, _dinv_of(dlo_ref[...]),
                     _dinv_of(dhi_ref[...]))
    t2pack = tlo_ref[0] + thi_ref[0]
    z = dmix * (a_ref[0] + t2pack) + b2_ref[...]
    zlo = jnp.where(lane < _L, z, 0.0)
    zhi = z - zlo
    giota = lax.broadcasted_iota(jnp.int32, (_BP, _G), 1)
    oh_lo = (blo_ref[...] == giota).astype(jnp.float32)
    oh_hi = (bhi_ref[...] == giota).astype(jnp.float32)
    dims = (((0,), (0,)), ((), ()))
    part = (lax.dot_general(oh_lo, zlo, dims,
                            preferred_element_type=jnp.float32,
                            precision=lax.Precision.HIGHEST) +
            lax.dot_general(oh_hi, zhi, dims,
                            preferred_element_type=jnp.float32,
                            precision=lax.Precision.HIGHEST))

    @pl.when(i == 0)
    def _():
        o_ref[...] = part

    @pl.when(i > 0)
    def _():
        o_ref[...] = o_ref[...] + part

    @pl.when(i == _PR // _BP - 1)
    def _():
        accv = o_ref[...]
        fold = (lax.broadcasted_iota(jnp.int32, (_C, _L), 0) % _L ==
                lax.broadcasted_iota(jnp.int32, (_C, _L), 1)
                ).astype(jnp.float32)
        sums = jnp.dot(accv, fold, preferred_element_type=jnp.float32,
                       precision=lax.Precision.HIGHEST)
        pooled = sums / jnp.maximum(cnt_ref[...], 1.0)
        o_ref[...] = jnp.concatenate(
            [pooled, jnp.zeros((_G, _C - _L), jnp.float32)], axis=1)


def _pool(acc2, t2, deg2, b2mix, batchp, cnt2):
    nhb = _PR // _BP // 2
    return pl.pallas_call(
        _pool_body,
        grid=(_PR // _BP,),
        in_specs=[
            pl.BlockSpec((1, _BP, _C), lambda i: (i // nhb, i % nhb, 0)),
            pl.BlockSpec((1, _BP, _C), lambda i: (0, i, 0)),
            pl.BlockSpec((1, _BP, _C), lambda i: (1, i + _PR // _BP, 0)),
            pl.BlockSpec((_BP, 1), lambda i: (i, 0)),
            pl.BlockSpec((_BP, 1), lambda i: (i + _PR // _BP, 0)),
            pl.BlockSpec((1, _C), lambda i: (0, 0)),
            pl.BlockSpec((_BP, 1), lambda i: (i, 0)),
            pl.BlockSpec((_BP, 1), lambda i: (i + _PR // _BP, 0)),
            pl.BlockSpec((_G, 1), lambda i: (0, 0)),
        ],
        out_specs=pl.BlockSpec((_G, _C), lambda i: (0, 0)),
        out_shape=jax.ShapeDtypeStruct((_G, _C), jnp.float32),
    )(acc2, t2, t2, deg2, deg2, b2mix, batchp, batchp, cnt2)


def kernel(x, edge_index, batch, W1, b1, W2, b2):
    npd = _EP - _E
    src_pad = (jnp.arange(npd, dtype=jnp.int32) * 97) % _N
    dst_pad = jnp.full((npd,), _NPAD - 1, jnp.int32)
    srcp = jnp.concatenate([edge_index[0], src_pad]).reshape(_NS, _NW, _CW)
    dstp = jnp.concatenate([edge_index[1], dst_pad]).reshape(_NS, _NW, _CW)
    xp = jnp.pad(x, ((0, _NPAD - _N), (0, 0)))
    batchp = jnp.concatenate(
        [batch, jnp.full((_NPAD - _N,), _G, jnp.int32)])
    bat_e = batchp.reshape(_NS, _BW, _CW)

    deg1, cnt = _deg_sc(dstp, bat_e)
    deg2 = deg1.reshape(_NPAD, 1)
    cnt2 = cnt.reshape(_G, 1)
    taba, tabb = _stage1(xp, W1[:, :_L], W1[:, _L:], deg2)
    acc1a = _agg_sc(taba.reshape(2 * _NPAD, _C), srcp, dstp)
    acc1b = _agg_sc(tabb.reshape(2 * _NPAD, _C), srcp, dstp)
    t2 = _stage2(acc1a, acc1b, taba, tabb, deg2,
                 b1.reshape(1, _H), W2)
    acc2 = _agg_sc(t2.reshape(2 * _NPAD, _C), srcp, dstp)
    b2mix = jnp.concatenate([b2, b2]).reshape(1, _C)
    pooled_aug = _pool(acc2, t2, deg2, b2mix,
                       batchp.reshape(_NPAD, 1), cnt2)
    return pooled_aug[:, :_L]
